# baseline (speedup 1.0000x reference)
import functools

import jax
import jax.numpy as jnp
import numpy as np
from jax.experimental import pallas as pl
from jax.experimental.pallas import tpu as pltpu
from jax.experimental.pallas import tpu_sc as plsc

N_NODES = 10000
C_IN = 128
HID = 64
W3 = 3 * HID
HW = W3 // 2
FEAT = 16
E_EDGES = 320000
SEQ = 5
BN_EPS = 1e-5
BLK = 1000

NC, NS, LANES = 2, 16, 16
NW = NC * NS
K_EDGE = 128
NCH0 = 94
NCH1 = 63
NCH_MAX = max(NCH0, NCH1)
E_CAP = NS * (NCH0 + NCH1) * K_EDGE
NPAD = 10112
NCHA = NPAD // K_EDGE
NPAD_DEG = 10240

_MESH = plsc.VectorSubcoreMesh(core_axis_name="c", subcore_axis_name="s")
_SC_PARAMS = pltpu.CompilerParams(use_tc_tiling_on_sc=False)


def _sigmoid(v):
    return 1.0 / (1.0 + jnp.exp(-v))


def _elu(v):
    return jnp.where(v > 0, v, jnp.exp(jnp.minimum(v, 0.0)) - 1.0)


@functools.partial(
    pl.kernel,
    out_type=jax.ShapeDtypeStruct((NC, NPAD_DEG), jnp.float32),
    mesh=_MESH,
    scratch_types=[
        pltpu.VMEM((NCH_MAX, K_EDGE), jnp.int32),
        pltpu.VMEM((NCH_MAX, K_EDGE), jnp.float32),
        pltpu.VMEM((640,), jnp.float32),
        pltpu.VMEM_SHARED((NPAD_DEG,), jnp.float32),
    ],
    compiler_params=_SC_PARAMS,
)
def _sc_deg(row_hbm, w_hbm, out_hbm, row_v, w_v, zbuf, acc):
    cid = jax.lax.axis_index("c")
    sid = jax.lax.axis_index("s")
    wid = sid * NC + cid
    pltpu.sync_copy(row_hbm.at[wid], row_v)
    pltpu.sync_copy(w_hbm.at[wid], w_v)

    def zb(i, carry):
        zbuf[pl.ds(i * LANES, LANES)] = jnp.zeros((LANES,), jnp.float32)
        return carry

    jax.lax.fori_loop(0, 640 // LANES, zb, None)
    pltpu.sync_copy(zbuf, acc.at[pl.ds(sid * 640, 640)])
    plsc.subcore_barrier()

    def body(j, carry):
        pltpu.sync_copy(w_v.at[j], acc.at[row_v.at[j]], add=True)
        return carry

    jax.lax.fori_loop(0, NCH_MAX, body, None)
    plsc.subcore_barrier()
    pltpu.sync_copy(acc.at[pl.ds(sid * 640, 640)],
                    out_hbm.at[cid, pl.ds(sid * 640, 640)])


@functools.partial(
    pl.kernel,
    out_type=[jax.ShapeDtypeStruct((NC, NPAD, HW), jnp.float32),
              jax.ShapeDtypeStruct((NC, NPAD, HW), jnp.float32)],
    mesh=_MESH,
    scratch_types=[
        pltpu.VMEM((NCH_MAX, K_EDGE), jnp.int32),
        pltpu.VMEM((NCH_MAX, K_EDGE), jnp.int32),
        pltpu.VMEM((NCH_MAX, K_EDGE), jnp.float32),
        pltpu.VMEM((K_EDGE, HW), jnp.float32),
        pltpu.VMEM_SHARED((NPAD, HW), jnp.float32),
        pltpu.SemaphoreType.DMA,
    ],
    compiler_params=_SC_PARAMS,
)
def _sc_prop(row_hbm, col_hbm, w_hbm, srca_hbm, srcb_hbm,
             parta_hbm, partb_hbm, row_v, col_v, w_v, rows, acc, sem):
    cid = jax.lax.axis_index("c")
    sid = jax.lax.axis_index("s")
    wid = sid * NC + cid
    pltpu.sync_copy(row_hbm.at[wid], row_v)
    pltpu.sync_copy(col_hbm.at[wid], col_v)
    pltpu.sync_copy(w_hbm.at[wid], w_v)
    nch = jnp.where(cid == 0, NCH0, NCH1)

    for src_hbm, part_hbm in ((srca_hbm, parta_hbm), (srcb_hbm, partb_hbm)):
        def zrow(l, carry):
            for c in range(HW // LANES):
                rows[l, pl.ds(c * LANES, LANES)] = jnp.zeros((LANES,),
                                                             jnp.float32)
            return carry

        jax.lax.fori_loop(0, K_EDGE, zrow, None)
        for jj in range(5):
            j = sid + jj * NS

            @pl.when(j < NCHA)
            def _():
                pltpu.sync_copy(rows, acc.at[pl.ds(j * K_EDGE, K_EDGE)])

        plsc.subcore_barrier()

        def pb(j, carry):
            pltpu.async_copy(src_hbm.at[row_v.at[j]], rows, sem).wait()

            def sb(lg, c2):
                w16 = w_v[j, pl.ds(lg * LANES, LANES)]
                for e in range(LANES):
                    we = jnp.full((LANES,), w16[e], jnp.float32)
                    l = lg * LANES + e
                    for c in range(HW // LANES):
                        sl = pl.ds(c * LANES, LANES)
                        rows[l, sl] = rows[l, sl] * we
                return c2

            jax.lax.fori_loop(0, K_EDGE // LANES, sb, None)
            pltpu.sync_copy(rows, acc.at[col_v.at[j]], add=True)
            return carry

        jax.lax.fori_loop(0, nch, pb, None)
        plsc.subcore_barrier()
        for jj in range(5):
            j = sid + jj * NS

            @pl.when(j < NCHA)
            def _():
                pltpu.sync_copy(acc.at[pl.ds(j * K_EDGE, K_EDGE)],
                                part_hbm.at[cid, pl.ds(j * K_EDGE, K_EDGE)])

        plsc.subcore_barrier()


def _tc_a_body(x_ref, degp_ref, wp_ref, wq_ref, wr_ref, b_ref,
               h0_ref, ua_ref, ub_ref, dis_ref):
    xs = [x_ref[s] for s in range(SEQ)]
    for t in range(3):
        xt = jnp.concatenate([xs[t], xs[t + 1], xs[t + 2]], axis=1)
        p = jnp.dot(xt, wp_ref[...], preferred_element_type=jnp.float32) + b_ref[0]
        q = jnp.dot(xt, wq_ref[...], preferred_element_type=jnp.float32) + b_ref[1]
        r = jnp.dot(xt, wr_ref[...], preferred_element_type=jnp.float32) + b_ref[2]
        h0_ref[:, t * HID:(t + 1) * HID] = jnp.maximum(p * _sigmoid(q) + r, 0.0)
    deg = degp_ref[:, 0] + degp_ref[:, 1]
    good = deg > 0
    dis = jnp.where(good, jax.lax.rsqrt(jnp.where(good, deg, 1.0)), 0.0)
    dis_ref[...] = dis[:, None]
    h0 = h0_ref[...]
    ua_ref[...] = dis[:, None] * h0[:, :HW]
    ub_ref[...] = dis[:, None] * h0[:, HW:]


def _tc_a(xs, degp, wp, wq, wr, b):
    grid = N_NODES // BLK
    return pl.pallas_call(
        _tc_a_body,
        grid=(grid,),
        in_specs=[
            pl.BlockSpec((SEQ, BLK, C_IN), lambda i: (0, i, 0)),
            pl.BlockSpec((BLK, 2), lambda i: (i, 0)),
            pl.BlockSpec((3 * C_IN, HID), lambda i: (0, 0)),
            pl.BlockSpec((3 * C_IN, HID), lambda i: (0, 0)),
            pl.BlockSpec((3 * C_IN, HID), lambda i: (0, 0)),
            pl.BlockSpec((3, HID), lambda i: (0, 0)),
        ],
        out_specs=[
            pl.BlockSpec((BLK, W3), lambda i: (i, 0)),
            pl.BlockSpec((BLK, HW), lambda i: (i, 0)),
            pl.BlockSpec((BLK, HW), lambda i: (i, 0)),
            pl.BlockSpec((BLK, 1), lambda i: (i, 0)),
        ],
        out_shape=[
            jax.ShapeDtypeStruct((N_NODES, W3), jnp.float32),
            jax.ShapeDtypeStruct((N_NODES, HW), jnp.float32),
            jax.ShapeDtypeStruct((N_NODES, HW), jnp.float32),
            jax.ShapeDtypeStruct((N_NODES, 1), jnp.float32),
        ],
    )(xs, degp, wp, wq, wr, b)


def _tc_mid_body(va_ref, vb_ref, dis_ref, t1_ref, ua_ref, ub_ref):
    dis = dis_ref[...]
    ya = va_ref[0] + va_ref[1]
    yb = vb_ref[0] + vb_ref[1]
    t1a = -dis * ya
    t1b = -dis * yb
    t1_ref[:, :HW] = t1a
    t1_ref[:, HW:] = t1b
    ua_ref[...] = dis * t1a
    ub_ref[...] = dis * t1b


def _tc_mid(va, vb, dis):
    grid = N_NODES // BLK
    return pl.pallas_call(
        _tc_mid_body,
        grid=(grid,),
        in_specs=[
            pl.BlockSpec((NC, BLK, HW), lambda i: (0, i, 0)),
            pl.BlockSpec((NC, BLK, HW), lambda i: (0, i, 0)),
            pl.BlockSpec((BLK, 1), lambda i: (i, 0)),
        ],
        out_specs=[
            pl.BlockSpec((BLK, W3), lambda i: (i, 0)),
            pl.BlockSpec((BLK, HW), lambda i: (i, 0)),
            pl.BlockSpec((BLK, HW), lambda i: (i, 0)),
        ],
        out_shape=[
            jax.ShapeDtypeStruct((N_NODES, W3), jnp.float32),
            jax.ShapeDtypeStruct((N_NODES, HW), jnp.float32),
            jax.ShapeDtypeStruct((N_NODES, HW), jnp.float32),
        ],
    )(va, vb, dis)


def _tc_b_body(h0_ref, t1_ref, va_ref, vb_ref, dis_ref, cw_ref, cb_ref,
               wp_ref, wq_ref, wr_ref, b2_ref, bn_ref,
               l1w_ref, l1b_ref, l2w_ref, l2b_ref, out_ref):
    h0 = h0_ref[...]
    t1 = t1_ref[...]
    dis = dis_ref[...]
    p2a = -dis * (va_ref[0] + va_ref[1])
    p2b = -dis * (vb_ref[0] + vb_ref[1])
    p2 = jnp.concatenate([p2a, p2b], axis=1)
    gs = []
    for t in range(3):
        sl = slice(t * HID, (t + 1) * HID)
        tx0 = h0[:, sl]
        tx2 = 2.0 * p2[:, sl] - tx0
        o = (jnp.dot(tx0, cw_ref[0], preferred_element_type=jnp.float32)
             + jnp.dot(t1[:, sl], cw_ref[1], preferred_element_type=jnp.float32)
             + jnp.dot(tx2, cw_ref[2], preferred_element_type=jnp.float32)
             + cb_ref[...])
        gs.append(jnp.maximum(o, 0.0))
    g = jnp.concatenate(gs, axis=1)
    p = jnp.dot(g, wp_ref[...], preferred_element_type=jnp.float32) + b2_ref[0]
    q = jnp.dot(g, wq_ref[...], preferred_element_type=jnp.float32) + b2_ref[1]
    r = jnp.dot(g, wr_ref[...], preferred_element_type=jnp.float32) + b2_ref[2]
    f = jnp.maximum(p * _sigmoid(q) + r, 0.0)
    f = f * bn_ref[:, 0:1] + bn_ref[:, 1:2]
    k = _elu(f)
    k = jnp.dot(k, l1w_ref[...], preferred_element_type=jnp.float32) + l1b_ref[...]
    k = _elu(k)
    out_ref[...] = jnp.dot(k, l2w_ref[...], preferred_element_type=jnp.float32) \
        + l2b_ref[...]


def _tc_b(h0, t1, va, vb, dis, cw, cb, wp, wq, wr, b2, bn, l1w, l1b, l2w, l2b):
    grid = N_NODES // BLK
    return pl.pallas_call(
        _tc_b_body,
        grid=(grid,),
        in_specs=[
            pl.BlockSpec((BLK, W3), lambda i: (i, 0)),
            pl.BlockSpec((BLK, W3), lambda i: (i, 0)),
            pl.BlockSpec((NC, BLK, HW), lambda i: (0, i, 0)),
            pl.BlockSpec((NC, BLK, HW), lambda i: (0, i, 0)),
            pl.BlockSpec((BLK, 1), lambda i: (i, 0)),
            pl.BlockSpec((3, HID, HID), lambda i: (0, 0, 0)),
            pl.BlockSpec((1, HID), lambda i: (0, 0)),
            pl.BlockSpec((W3, FEAT), lambda i: (0, 0)),
            pl.BlockSpec((W3, FEAT), lambda i: (0, 0)),
            pl.BlockSpec((W3, FEAT), lambda i: (0, 0)),
            pl.BlockSpec((3, FEAT), lambda i: (0, 0)),
            pl.BlockSpec((BLK, 2), lambda i: (i, 0)),
            pl.BlockSpec((FEAT, 10), lambda i: (0, 0)),
            pl.BlockSpec((1, 10), lambda i: (0, 0)),
            pl.BlockSpec((10, 1), lambda i: (0, 0)),
            pl.BlockSpec((1, 1), lambda i: (0, 0)),
        ],
        out_specs=pl.BlockSpec((BLK, 1), lambda i: (i, 0)),
        out_shape=jax.ShapeDtypeStruct((N_NODES, 1), jnp.float32),
    )(h0, t1, va, vb, dis, cw, cb, wp, wq, wr, b2, bn, l1w, l1b, l2w, l2b)


def kernel(x, edge_index, edge_weight, params):
    row, col = edge_index[0], edge_index[1]
    xs = x[0]

    def tconv_w(w):
        return jnp.transpose(w[:, :, 0, :], (2, 1, 0)).reshape(w.shape[1] * 3,
                                                               w.shape[0])

    wp1, wq1, wr1 = (tconv_w(w) for w in params["tc1_w"])
    b1 = jnp.stack(params["tc1_b"])
    wp2, wq2, wr2 = (tconv_w(w) for w in params["tc2_w"])
    b2 = jnp.stack(params["tc2_b"])
    cw = jnp.stack(params["cheb_w"])
    cb = params["cheb_b"][None, :]
    bn_scale = params["bn_w"] * np.float32(1.0 / np.sqrt(1.0 + BN_EPS))
    bn = jnp.stack([bn_scale, params["bn_b"]], axis=1)
    l1w = params["lin1_w"].T
    l1b = params["lin1_b"][None, :]
    l2w = params["lin2_w"].T
    l2b = params["lin2_b"][None, :]

    spread = (jnp.arange(NCH_MAX * K_EDGE, dtype=jnp.int32) * 8) % N_NODES

    def shard_edges(a, idx_pad):
        pad = spread[:E_CAP - E_EDGES].astype(a.dtype) if idx_pad \
            else jnp.zeros((E_CAP - E_EDGES,), a.dtype)
        a = jnp.concatenate([a, pad])
        segs, off = [], 0
        for w in range(NW):
            n = NCH0 if (w % NC == 0) else NCH1
            s = n * K_EDGE
            seg = a[off:off + s]
            off += s
            if n < NCH_MAX:
                tail = spread[:(NCH_MAX - n) * K_EDGE].astype(a.dtype) \
                    if idx_pad else jnp.zeros(((NCH_MAX - n) * K_EDGE,),
                                              a.dtype)
                seg = jnp.concatenate([seg, tail])
            segs.append(seg)
        return jnp.stack(segs).reshape(NW, NCH_MAX, K_EDGE)

    row_p = shard_edges(row, True)
    col_p = shard_edges(col, True)
    w_p = shard_edges(edge_weight, False)

    degp = _sc_deg(row_p, w_p)
    h0, u1a, u1b, dis = _tc_a(xs, degp.T, wp1, wq1, wr1, b1)

    v1a, v1b = _sc_prop(row_p, col_p, w_p, u1a, u1b)
    t1, u2a, u2b = _tc_mid(v1a, v1b, dis)
    v2a, v2b = _sc_prop(row_p, col_p, w_p, u2a, u2b)

    return _tc_b(h0, t1, v2a, v2b, dis, cw, cb, wp2, wq2, wr2, b2, bn,
                 l1w, l1b, l2w, l2b)


# speedup vs baseline: 1.1448x; 1.1448x over previous
import functools

import jax
import jax.numpy as jnp
import numpy as np
from jax.experimental import pallas as pl
from jax.experimental.pallas import tpu as pltpu
from jax.experimental.pallas import tpu_sc as plsc

N_NODES = 10000
C_IN = 128
HID = 64
W3 = 3 * HID
HW = W3 // 2
FEAT = 16
E_EDGES = 320000
SEQ = 5
BN_EPS = 1e-5
BLK = 1000

NC, NS, LANES = 2, 16, 16
NW = NC * NS
K_EDGE = 256
NCH0 = 47
NCH1 = 32
NCH_MAX = max(NCH0, NCH1)
E_CAP = NS * (NCH0 + NCH1) * K_EDGE
NPAD = 10112
KACC = 128
NCHA = NPAD // KACC
NPAD_DEG = 10240

_MESH = plsc.VectorSubcoreMesh(core_axis_name="c", subcore_axis_name="s")
_SC_PARAMS = pltpu.CompilerParams(use_tc_tiling_on_sc=False)


def _sigmoid(v):
    return 1.0 / (1.0 + jnp.exp(-v))


def _elu(v):
    return jnp.where(v > 0, v, jnp.exp(jnp.minimum(v, 0.0)) - 1.0)


@functools.partial(
    pl.kernel,
    out_type=jax.ShapeDtypeStruct((NC, NPAD_DEG), jnp.float32),
    mesh=_MESH,
    scratch_types=[
        pltpu.VMEM((NCH_MAX, K_EDGE), jnp.int32),
        pltpu.VMEM((NCH_MAX, K_EDGE), jnp.float32),
        pltpu.VMEM((640,), jnp.float32),
        pltpu.VMEM_SHARED((NPAD_DEG,), jnp.float32),
    ],
    compiler_params=_SC_PARAMS,
)
def _sc_deg(row_hbm, w_hbm, out_hbm, row_v, w_v, zbuf, acc):
    cid = jax.lax.axis_index("c")
    sid = jax.lax.axis_index("s")
    wid = sid * NC + cid
    pltpu.sync_copy(row_hbm.at[wid], row_v)
    pltpu.sync_copy(w_hbm.at[wid], w_v)

    def zb(i, carry):
        zbuf[pl.ds(i * LANES, LANES)] = jnp.zeros((LANES,), jnp.float32)
        return carry

    jax.lax.fori_loop(0, 640 // LANES, zb, None)
    pltpu.sync_copy(zbuf, acc.at[pl.ds(sid * 640, 640)])
    plsc.subcore_barrier()

    def body(j, carry):
        pltpu.sync_copy(w_v.at[j], acc.at[row_v.at[j]], add=True)
        return carry

    jax.lax.fori_loop(0, NCH_MAX, body, None)
    plsc.subcore_barrier()
    pltpu.sync_copy(acc.at[pl.ds(sid * 640, 640)],
                    out_hbm.at[cid, pl.ds(sid * 640, 640)])


@functools.partial(
    pl.kernel,
    out_type=jax.ShapeDtypeStruct((NC, NPAD, HW), jnp.float32),
    mesh=_MESH,
    scratch_types=[
        pltpu.VMEM((NCH_MAX, K_EDGE), jnp.int32),
        pltpu.VMEM((NCH_MAX, K_EDGE), jnp.int32),
        pltpu.VMEM((NCH_MAX, K_EDGE), jnp.float32),
        pltpu.VMEM((K_EDGE, HW), jnp.float32),
        pltpu.VMEM_SHARED((NPAD, HW), jnp.float32),
        pltpu.SemaphoreType.DMA,
    ],
    compiler_params=_SC_PARAMS,
)
def _sc_prop(row_hbm, col_hbm, w_hbm, src_hbm, part_hbm,
             row_v, col_v, w_v, rows, acc, sem):
    cid = jax.lax.axis_index("c")
    sid = jax.lax.axis_index("s")
    wid = sid * NC + cid
    pltpu.sync_copy(row_hbm.at[wid], row_v)
    pltpu.sync_copy(col_hbm.at[wid], col_v)
    pltpu.sync_copy(w_hbm.at[wid], w_v)

    def zrow(l, carry):
        for c in range(HW // LANES):
            rows[l, pl.ds(c * LANES, LANES)] = jnp.zeros((LANES,), jnp.float32)
        return carry

    jax.lax.fori_loop(0, KACC, zrow, None)
    for jj in range(5):
        j = sid + jj * NS

        @pl.when(j < NCHA)
        def _():
            pltpu.sync_copy(rows.at[pl.ds(0, KACC)],
                            acc.at[pl.ds(j * KACC, KACC)])

    plsc.subcore_barrier()

    def pb(j, carry):
        pltpu.async_copy(src_hbm.at[row_v.at[j]], rows, sem).wait()

        def sb(lg, c2):
            w16 = w_v[j, pl.ds(lg * LANES, LANES)]
            for e in range(LANES):
                we = jnp.full((LANES,), w16[e], jnp.float32)
                l = lg * LANES + e
                for c in range(HW // LANES):
                    sl = pl.ds(c * LANES, LANES)
                    rows[l, sl] = rows[l, sl] * we
            return c2

        jax.lax.fori_loop(0, K_EDGE // LANES, sb, None)
        pltpu.sync_copy(rows, acc.at[col_v.at[j]], add=True)
        return carry

    nch = jnp.where(cid == 0, NCH0, NCH1)
    jax.lax.fori_loop(0, nch, pb, None)
    plsc.subcore_barrier()
    for jj in range(5):
        j = sid + jj * NS

        @pl.when(j < NCHA)
        def _():
            pltpu.sync_copy(acc.at[pl.ds(j * KACC, KACC)],
                            part_hbm.at[cid, pl.ds(j * KACC, KACC)])


def _tc_a_body(x_ref, degp_ref, wp_ref, wq_ref, wr_ref, b_ref,
               h0_ref, ua_ref, ub_ref, dis_ref):
    xs = [x_ref[s] for s in range(SEQ)]
    for t in range(3):
        xt = jnp.concatenate([xs[t], xs[t + 1], xs[t + 2]], axis=1)
        p = jnp.dot(xt, wp_ref[...], preferred_element_type=jnp.float32) + b_ref[0]
        q = jnp.dot(xt, wq_ref[...], preferred_element_type=jnp.float32) + b_ref[1]
        r = jnp.dot(xt, wr_ref[...], preferred_element_type=jnp.float32) + b_ref[2]
        h0_ref[:, t * HID:(t + 1) * HID] = jnp.maximum(p * _sigmoid(q) + r, 0.0)
    deg = degp_ref[:, 0] + degp_ref[:, 1]
    good = deg > 0
    dis = jnp.where(good, jax.lax.rsqrt(jnp.where(good, deg, 1.0)), 0.0)
    dis_ref[...] = dis[:, None]
    h0 = h0_ref[...]
    ua_ref[...] = dis[:, None] * h0[:, :HW]
    ub_ref[...] = dis[:, None] * h0[:, HW:]


def _tc_a(xs, degp, wp, wq, wr, b):
    grid = N_NODES // BLK
    return pl.pallas_call(
        _tc_a_body,
        grid=(grid,),
        in_specs=[
            pl.BlockSpec((SEQ, BLK, C_IN), lambda i: (0, i, 0)),
            pl.BlockSpec((BLK, 2), lambda i: (i, 0)),
            pl.BlockSpec((3 * C_IN, HID), lambda i: (0, 0)),
            pl.BlockSpec((3 * C_IN, HID), lambda i: (0, 0)),
            pl.BlockSpec((3 * C_IN, HID), lambda i: (0, 0)),
            pl.BlockSpec((3, HID), lambda i: (0, 0)),
        ],
        out_specs=[
            pl.BlockSpec((BLK, W3), lambda i: (i, 0)),
            pl.BlockSpec((BLK, HW), lambda i: (i, 0)),
            pl.BlockSpec((BLK, HW), lambda i: (i, 0)),
            pl.BlockSpec((BLK, 1), lambda i: (i, 0)),
        ],
        out_shape=[
            jax.ShapeDtypeStruct((N_NODES, W3), jnp.float32),
            jax.ShapeDtypeStruct((N_NODES, HW), jnp.float32),
            jax.ShapeDtypeStruct((N_NODES, HW), jnp.float32),
            jax.ShapeDtypeStruct((N_NODES, 1), jnp.float32),
        ],
    )(xs, degp, wp, wq, wr, b)


def _tc_mid_body(va_ref, vb_ref, dis_ref, t1_ref, ua_ref, ub_ref):
    dis = dis_ref[...]
    ya = va_ref[0] + va_ref[1]
    yb = vb_ref[0] + vb_ref[1]
    t1a = -dis * ya
    t1b = -dis * yb
    t1_ref[:, :HW] = t1a
    t1_ref[:, HW:] = t1b
    ua_ref[...] = dis * t1a
    ub_ref[...] = dis * t1b


def _tc_mid(va, vb, dis):
    grid = N_NODES // BLK
    return pl.pallas_call(
        _tc_mid_body,
        grid=(grid,),
        in_specs=[
            pl.BlockSpec((NC, BLK, HW), lambda i: (0, i, 0)),
            pl.BlockSpec((NC, BLK, HW), lambda i: (0, i, 0)),
            pl.BlockSpec((BLK, 1), lambda i: (i, 0)),
        ],
        out_specs=[
            pl.BlockSpec((BLK, W3), lambda i: (i, 0)),
            pl.BlockSpec((BLK, HW), lambda i: (i, 0)),
            pl.BlockSpec((BLK, HW), lambda i: (i, 0)),
        ],
        out_shape=[
            jax.ShapeDtypeStruct((N_NODES, W3), jnp.float32),
            jax.ShapeDtypeStruct((N_NODES, HW), jnp.float32),
            jax.ShapeDtypeStruct((N_NODES, HW), jnp.float32),
        ],
    )(va, vb, dis)


def _tc_b_body(h0_ref, t1_ref, va_ref, vb_ref, dis_ref, cw_ref, cb_ref,
               wp_ref, wq_ref, wr_ref, b2_ref, bn_ref,
               l1w_ref, l1b_ref, l2w_ref, l2b_ref, out_ref):
    h0 = h0_ref[...]
    t1 = t1_ref[...]
    dis = dis_ref[...]
    p2a = -dis * (va_ref[0] + va_ref[1])
    p2b = -dis * (vb_ref[0] + vb_ref[1])
    p2 = jnp.concatenate([p2a, p2b], axis=1)
    gs = []
    for t in range(3):
        sl = slice(t * HID, (t + 1) * HID)
        tx0 = h0[:, sl]
        tx2 = 2.0 * p2[:, sl] - tx0
        o = (jnp.dot(tx0, cw_ref[0], preferred_element_type=jnp.float32)
             + jnp.dot(t1[:, sl], cw_ref[1], preferred_element_type=jnp.float32)
             + jnp.dot(tx2, cw_ref[2], preferred_element_type=jnp.float32)
             + cb_ref[...])
        gs.append(jnp.maximum(o, 0.0))
    g = jnp.concatenate(gs, axis=1)
    p = jnp.dot(g, wp_ref[...], preferred_element_type=jnp.float32) + b2_ref[0]
    q = jnp.dot(g, wq_ref[...], preferred_element_type=jnp.float32) + b2_ref[1]
    r = jnp.dot(g, wr_ref[...], preferred_element_type=jnp.float32) + b2_ref[2]
    f = jnp.maximum(p * _sigmoid(q) + r, 0.0)
    f = f * bn_ref[:, 0:1] + bn_ref[:, 1:2]
    k = _elu(f)
    k = jnp.dot(k, l1w_ref[...], preferred_element_type=jnp.float32) + l1b_ref[...]
    k = _elu(k)
    out_ref[...] = jnp.dot(k, l2w_ref[...], preferred_element_type=jnp.float32) \
        + l2b_ref[...]


def _tc_b(h0, t1, va, vb, dis, cw, cb, wp, wq, wr, b2, bn, l1w, l1b, l2w, l2b):
    grid = N_NODES // BLK
    return pl.pallas_call(
        _tc_b_body,
        grid=(grid,),
        in_specs=[
            pl.BlockSpec((BLK, W3), lambda i: (i, 0)),
            pl.BlockSpec((BLK, W3), lambda i: (i, 0)),
            pl.BlockSpec((NC, BLK, HW), lambda i: (0, i, 0)),
            pl.BlockSpec((NC, BLK, HW), lambda i: (0, i, 0)),
            pl.BlockSpec((BLK, 1), lambda i: (i, 0)),
            pl.BlockSpec((3, HID, HID), lambda i: (0, 0, 0)),
            pl.BlockSpec((1, HID), lambda i: (0, 0)),
            pl.BlockSpec((W3, FEAT), lambda i: (0, 0)),
            pl.BlockSpec((W3, FEAT), lambda i: (0, 0)),
            pl.BlockSpec((W3, FEAT), lambda i: (0, 0)),
            pl.BlockSpec((3, FEAT), lambda i: (0, 0)),
            pl.BlockSpec((BLK, 2), lambda i: (i, 0)),
            pl.BlockSpec((FEAT, 10), lambda i: (0, 0)),
            pl.BlockSpec((1, 10), lambda i: (0, 0)),
            pl.BlockSpec((10, 1), lambda i: (0, 0)),
            pl.BlockSpec((1, 1), lambda i: (0, 0)),
        ],
        out_specs=pl.BlockSpec((BLK, 1), lambda i: (i, 0)),
        out_shape=jax.ShapeDtypeStruct((N_NODES, 1), jnp.float32),
    )(h0, t1, va, vb, dis, cw, cb, wp, wq, wr, b2, bn, l1w, l1b, l2w, l2b)


def kernel(x, edge_index, edge_weight, params):
    row, col = edge_index[0], edge_index[1]
    xs = x[0]

    def tconv_w(w):
        return jnp.transpose(w[:, :, 0, :], (2, 1, 0)).reshape(w.shape[1] * 3,
                                                               w.shape[0])

    wp1, wq1, wr1 = (tconv_w(w) for w in params["tc1_w"])
    b1 = jnp.stack(params["tc1_b"])
    wp2, wq2, wr2 = (tconv_w(w) for w in params["tc2_w"])
    b2 = jnp.stack(params["tc2_b"])
    cw = jnp.stack(params["cheb_w"])
    cb = params["cheb_b"][None, :]
    bn_scale = params["bn_w"] * np.float32(1.0 / np.sqrt(1.0 + BN_EPS))
    bn = jnp.stack([bn_scale, params["bn_b"]], axis=1)
    l1w = params["lin1_w"].T
    l1b = params["lin1_b"][None, :]
    l2w = params["lin2_w"].T
    l2b = params["lin2_b"][None, :]

    spread = (jnp.arange(NCH_MAX * K_EDGE, dtype=jnp.int32) * 8) % N_NODES

    def shard_edges(a, idx_pad):
        pad = spread[:E_CAP - E_EDGES].astype(a.dtype) if idx_pad \
            else jnp.zeros((E_CAP - E_EDGES,), a.dtype)
        a = jnp.concatenate([a, pad])
        segs, off = [], 0
        for w in range(NW):
            n = NCH0 if (w % NC == 0) else NCH1
            s = n * K_EDGE
            seg = a[off:off + s]
            off += s
            if n < NCH_MAX:
                tail = spread[:(NCH_MAX - n) * K_EDGE].astype(a.dtype) \
                    if idx_pad else jnp.zeros(((NCH_MAX - n) * K_EDGE,),
                                              a.dtype)
                seg = jnp.concatenate([seg, tail])
            segs.append(seg)
        return jnp.stack(segs).reshape(NW, NCH_MAX, K_EDGE)

    row_p = shard_edges(row, True)
    col_p = shard_edges(col, True)
    w_p = shard_edges(edge_weight, False)

    degp = _sc_deg(row_p, w_p)
    h0, u1a, u1b, dis = _tc_a(xs, degp.T, wp1, wq1, wr1, b1)

    v1a = _sc_prop(row_p, col_p, w_p, u1a)
    v1b = _sc_prop(row_p, col_p, w_p, u1b)
    t1, u2a, u2b = _tc_mid(v1a, v1b, dis)
    v2a = _sc_prop(row_p, col_p, w_p, u2a)
    v2b = _sc_prop(row_p, col_p, w_p, u2b)

    return _tc_b(h0, t1, v2a, v2b, dis, cw, cb, wp2, wq2, wr2, b2, bn,
                 l1w, l1b, l2w, l2b)


# speedup vs baseline: 1.2915x; 1.1282x over previous
import functools

import jax
import jax.numpy as jnp
import numpy as np
from jax.experimental import pallas as pl
from jax.experimental.pallas import tpu as pltpu
from jax.experimental.pallas import tpu_sc as plsc

N_NODES = 10000
C_IN = 128
HID = 64
W3 = 3 * HID
HW = W3 // 2
FEAT = 16
E_EDGES = 320000
SEQ = 5
BN_EPS = 1e-5
BLK = 1000

NC, NS, LANES = 2, 16, 16
NW = NC * NS
K_EDGE = 256
NCH0 = 40
NCH1 = 39
NCH_MAX = max(NCH0, NCH1)
E_CAP = NS * (NCH0 + NCH1) * K_EDGE
NPAD = 10112
KACC = 128
NCHA = NPAD // KACC
NPAD_DEG = 10240

_MESH = plsc.VectorSubcoreMesh(core_axis_name="c", subcore_axis_name="s")
_SC_PARAMS = pltpu.CompilerParams(use_tc_tiling_on_sc=False)


def _sigmoid(v):
    return 1.0 / (1.0 + jnp.exp(-v))


def _elu(v):
    return jnp.where(v > 0, v, jnp.exp(jnp.minimum(v, 0.0)) - 1.0)


@functools.partial(
    pl.kernel,
    out_type=jax.ShapeDtypeStruct((NC, NPAD_DEG), jnp.float32),
    mesh=_MESH,
    scratch_types=[
        pltpu.VMEM((NCH_MAX, K_EDGE), jnp.int32),
        pltpu.VMEM((NCH_MAX, K_EDGE), jnp.float32),
        pltpu.VMEM((640,), jnp.float32),
        pltpu.VMEM_SHARED((NPAD_DEG,), jnp.float32),
    ],
    compiler_params=_SC_PARAMS,
)
def _sc_deg(row_hbm, w_hbm, out_hbm, row_v, w_v, zbuf, acc):
    cid = jax.lax.axis_index("c")
    sid = jax.lax.axis_index("s")
    wid = sid * NC + cid
    pltpu.sync_copy(row_hbm.at[wid], row_v)
    pltpu.sync_copy(w_hbm.at[wid], w_v)

    def zb(i, carry):
        zbuf[pl.ds(i * LANES, LANES)] = jnp.zeros((LANES,), jnp.float32)
        return carry

    jax.lax.fori_loop(0, 640 // LANES, zb, None)
    pltpu.sync_copy(zbuf, acc.at[pl.ds(sid * 640, 640)])
    plsc.subcore_barrier()

    def body(j, carry):
        pltpu.sync_copy(w_v.at[j], acc.at[row_v.at[j]], add=True)
        return carry

    jax.lax.fori_loop(0, NCH_MAX, body, None)
    plsc.subcore_barrier()
    pltpu.sync_copy(acc.at[pl.ds(sid * 640, 640)],
                    out_hbm.at[cid, pl.ds(sid * 640, 640)])


@functools.partial(
    pl.kernel,
    out_type=jax.ShapeDtypeStruct((NC, NPAD, HW), jnp.float32),
    mesh=_MESH,
    scratch_types=[
        pltpu.VMEM((NCH_MAX, K_EDGE), jnp.int32),
        pltpu.VMEM((NCH_MAX, K_EDGE), jnp.int32),
        pltpu.VMEM((NCH_MAX, K_EDGE), jnp.float32),
        pltpu.VMEM((K_EDGE, HW), jnp.float32),
        pltpu.VMEM_SHARED((NPAD, HW), jnp.float32),
        pltpu.SemaphoreType.DMA,
    ],
    compiler_params=_SC_PARAMS,
)
def _sc_prop(row_hbm, col_hbm, w_hbm, src_hbm, part_hbm,
             row_v, col_v, w_v, rows, acc, sem):
    cid = jax.lax.axis_index("c")
    sid = jax.lax.axis_index("s")
    wid = sid * NC + cid
    pltpu.sync_copy(row_hbm.at[wid], row_v)
    pltpu.sync_copy(col_hbm.at[wid], col_v)
    pltpu.sync_copy(w_hbm.at[wid], w_v)

    def zrow(l, carry):
        for c in range(HW // LANES):
            rows[l, pl.ds(c * LANES, LANES)] = jnp.zeros((LANES,), jnp.float32)
        return carry

    jax.lax.fori_loop(0, KACC, zrow, None)
    for jj in range(5):
        j = sid + jj * NS

        @pl.when(j < NCHA)
        def _():
            pltpu.sync_copy(rows.at[pl.ds(0, KACC)],
                            acc.at[pl.ds(j * KACC, KACC)])

    plsc.subcore_barrier()

    def pb(j, carry):
        pltpu.async_copy(src_hbm.at[row_v.at[j]], rows, sem).wait()

        def sb(lg, c2):
            w16 = w_v[j, pl.ds(lg * LANES, LANES)]
            for e in range(LANES):
                we = jnp.full((LANES,), w16[e], jnp.float32)
                l = lg * LANES + e
                for c in range(HW // LANES):
                    sl = pl.ds(c * LANES, LANES)
                    rows[l, sl] = rows[l, sl] * we
            return c2

        jax.lax.fori_loop(0, K_EDGE // LANES, sb, None)
        pltpu.sync_copy(rows, acc.at[col_v.at[j]], add=True)
        return carry

    nch = jnp.where(cid == 0, NCH0, NCH1)
    jax.lax.fori_loop(0, nch, pb, None)
    plsc.subcore_barrier()
    for jj in range(5):
        j = sid + jj * NS

        @pl.when(j < NCHA)
        def _():
            pltpu.sync_copy(acc.at[pl.ds(j * KACC, KACC)],
                            part_hbm.at[cid, pl.ds(j * KACC, KACC)])


def _tc_a_body(x_ref, degp_ref, wp_ref, wq_ref, wr_ref, b_ref,
               h0_ref, ua_ref, ub_ref, dis_ref):
    xs = [x_ref[s] for s in range(SEQ)]
    for t in range(3):
        xt = jnp.concatenate([xs[t], xs[t + 1], xs[t + 2]], axis=1)
        p = jnp.dot(xt, wp_ref[...], preferred_element_type=jnp.float32) + b_ref[0]
        q = jnp.dot(xt, wq_ref[...], preferred_element_type=jnp.float32) + b_ref[1]
        r = jnp.dot(xt, wr_ref[...], preferred_element_type=jnp.float32) + b_ref[2]
        h0_ref[:, t * HID:(t + 1) * HID] = jnp.maximum(p * _sigmoid(q) + r, 0.0)
    deg = degp_ref[:, 0] + degp_ref[:, 1]
    good = deg > 0
    dis = jnp.where(good, jax.lax.rsqrt(jnp.where(good, deg, 1.0)), 0.0)
    dis_ref[...] = dis[:, None]
    h0 = h0_ref[...]
    ua_ref[...] = dis[:, None] * h0[:, :HW]
    ub_ref[...] = dis[:, None] * h0[:, HW:]


def _tc_a(xs, degp, wp, wq, wr, b):
    grid = N_NODES // BLK
    return pl.pallas_call(
        _tc_a_body,
        grid=(grid,),
        in_specs=[
            pl.BlockSpec((SEQ, BLK, C_IN), lambda i: (0, i, 0)),
            pl.BlockSpec((BLK, 2), lambda i: (i, 0)),
            pl.BlockSpec((3 * C_IN, HID), lambda i: (0, 0)),
            pl.BlockSpec((3 * C_IN, HID), lambda i: (0, 0)),
            pl.BlockSpec((3 * C_IN, HID), lambda i: (0, 0)),
            pl.BlockSpec((3, HID), lambda i: (0, 0)),
        ],
        out_specs=[
            pl.BlockSpec((BLK, W3), lambda i: (i, 0)),
            pl.BlockSpec((BLK, HW), lambda i: (i, 0)),
            pl.BlockSpec((BLK, HW), lambda i: (i, 0)),
            pl.BlockSpec((BLK, 1), lambda i: (i, 0)),
        ],
        out_shape=[
            jax.ShapeDtypeStruct((N_NODES, W3), jnp.float32),
            jax.ShapeDtypeStruct((N_NODES, HW), jnp.float32),
            jax.ShapeDtypeStruct((N_NODES, HW), jnp.float32),
            jax.ShapeDtypeStruct((N_NODES, 1), jnp.float32),
        ],
    )(xs, degp, wp, wq, wr, b)


def _tc_mid_body(va_ref, vb_ref, dis_ref, t1_ref, ua_ref, ub_ref):
    dis = dis_ref[...]
    ya = va_ref[0] + va_ref[1]
    yb = vb_ref[0] + vb_ref[1]
    t1a = -dis * ya
    t1b = -dis * yb
    t1_ref[:, :HW] = t1a
    t1_ref[:, HW:] = t1b
    ua_ref[...] = dis * t1a
    ub_ref[...] = dis * t1b


def _tc_mid(va, vb, dis):
    grid = N_NODES // BLK
    return pl.pallas_call(
        _tc_mid_body,
        grid=(grid,),
        in_specs=[
            pl.BlockSpec((NC, BLK, HW), lambda i: (0, i, 0)),
            pl.BlockSpec((NC, BLK, HW), lambda i: (0, i, 0)),
            pl.BlockSpec((BLK, 1), lambda i: (i, 0)),
        ],
        out_specs=[
            pl.BlockSpec((BLK, W3), lambda i: (i, 0)),
            pl.BlockSpec((BLK, HW), lambda i: (i, 0)),
            pl.BlockSpec((BLK, HW), lambda i: (i, 0)),
        ],
        out_shape=[
            jax.ShapeDtypeStruct((N_NODES, W3), jnp.float32),
            jax.ShapeDtypeStruct((N_NODES, HW), jnp.float32),
            jax.ShapeDtypeStruct((N_NODES, HW), jnp.float32),
        ],
    )(va, vb, dis)


def _tc_b_body(h0_ref, t1_ref, va_ref, vb_ref, dis_ref, cw_ref, cb_ref,
               wp_ref, wq_ref, wr_ref, b2_ref, bn_ref,
               l1w_ref, l1b_ref, l2w_ref, l2b_ref, out_ref):
    h0 = h0_ref[...]
    t1 = t1_ref[...]
    dis = dis_ref[...]
    p2a = -dis * (va_ref[0] + va_ref[1])
    p2b = -dis * (vb_ref[0] + vb_ref[1])
    p2 = jnp.concatenate([p2a, p2b], axis=1)
    gs = []
    for t in range(3):
        sl = slice(t * HID, (t + 1) * HID)
        tx0 = h0[:, sl]
        tx2 = 2.0 * p2[:, sl] - tx0
        o = (jnp.dot(tx0, cw_ref[0], preferred_element_type=jnp.float32)
             + jnp.dot(t1[:, sl], cw_ref[1], preferred_element_type=jnp.float32)
             + jnp.dot(tx2, cw_ref[2], preferred_element_type=jnp.float32)
             + cb_ref[...])
        gs.append(jnp.maximum(o, 0.0))
    g = jnp.concatenate(gs, axis=1)
    p = jnp.dot(g, wp_ref[...], preferred_element_type=jnp.float32) + b2_ref[0]
    q = jnp.dot(g, wq_ref[...], preferred_element_type=jnp.float32) + b2_ref[1]
    r = jnp.dot(g, wr_ref[...], preferred_element_type=jnp.float32) + b2_ref[2]
    f = jnp.maximum(p * _sigmoid(q) + r, 0.0)
    f = f * bn_ref[:, 0:1] + bn_ref[:, 1:2]
    k = _elu(f)
    k = jnp.dot(k, l1w_ref[...], preferred_element_type=jnp.float32) + l1b_ref[...]
    k = _elu(k)
    out_ref[...] = jnp.dot(k, l2w_ref[...], preferred_element_type=jnp.float32) \
        + l2b_ref[...]


def _tc_b(h0, t1, va, vb, dis, cw, cb, wp, wq, wr, b2, bn, l1w, l1b, l2w, l2b):
    grid = N_NODES // BLK
    return pl.pallas_call(
        _tc_b_body,
        grid=(grid,),
        in_specs=[
            pl.BlockSpec((BLK, W3), lambda i: (i, 0)),
            pl.BlockSpec((BLK, W3), lambda i: (i, 0)),
            pl.BlockSpec((NC, BLK, HW), lambda i: (0, i, 0)),
            pl.BlockSpec((NC, BLK, HW), lambda i: (0, i, 0)),
            pl.BlockSpec((BLK, 1), lambda i: (i, 0)),
            pl.BlockSpec((3, HID, HID), lambda i: (0, 0, 0)),
            pl.BlockSpec((1, HID), lambda i: (0, 0)),
            pl.BlockSpec((W3, FEAT), lambda i: (0, 0)),
            pl.BlockSpec((W3, FEAT), lambda i: (0, 0)),
            pl.BlockSpec((W3, FEAT), lambda i: (0, 0)),
            pl.BlockSpec((3, FEAT), lambda i: (0, 0)),
            pl.BlockSpec((BLK, 2), lambda i: (i, 0)),
            pl.BlockSpec((FEAT, 10), lambda i: (0, 0)),
            pl.BlockSpec((1, 10), lambda i: (0, 0)),
            pl.BlockSpec((10, 1), lambda i: (0, 0)),
            pl.BlockSpec((1, 1), lambda i: (0, 0)),
        ],
        out_specs=pl.BlockSpec((BLK, 1), lambda i: (i, 0)),
        out_shape=jax.ShapeDtypeStruct((N_NODES, 1), jnp.float32),
    )(h0, t1, va, vb, dis, cw, cb, wp, wq, wr, b2, bn, l1w, l1b, l2w, l2b)


def kernel(x, edge_index, edge_weight, params):
    row, col = edge_index[0], edge_index[1]
    xs = x[0]

    def tconv_w(w):
        return jnp.transpose(w[:, :, 0, :], (2, 1, 0)).reshape(w.shape[1] * 3,
                                                               w.shape[0])

    wp1, wq1, wr1 = (tconv_w(w) for w in params["tc1_w"])
    b1 = jnp.stack(params["tc1_b"])
    wp2, wq2, wr2 = (tconv_w(w) for w in params["tc2_w"])
    b2 = jnp.stack(params["tc2_b"])
    cw = jnp.stack(params["cheb_w"])
    cb = params["cheb_b"][None, :]
    bn_scale = params["bn_w"] * np.float32(1.0 / np.sqrt(1.0 + BN_EPS))
    bn = jnp.stack([bn_scale, params["bn_b"]], axis=1)
    l1w = params["lin1_w"].T
    l1b = params["lin1_b"][None, :]
    l2w = params["lin2_w"].T
    l2b = params["lin2_b"][None, :]

    spread = (jnp.arange(NCH_MAX * K_EDGE, dtype=jnp.int32) * 8) % N_NODES

    def shard_edges(a, idx_pad):
        pad = spread[:E_CAP - E_EDGES].astype(a.dtype) if idx_pad \
            else jnp.zeros((E_CAP - E_EDGES,), a.dtype)
        a = jnp.concatenate([a, pad])
        segs, off = [], 0
        for w in range(NW):
            n = NCH0 if (w % NC == 0) else NCH1
            s = n * K_EDGE
            seg = a[off:off + s]
            off += s
            if n < NCH_MAX:
                tail = spread[:(NCH_MAX - n) * K_EDGE].astype(a.dtype) \
                    if idx_pad else jnp.zeros(((NCH_MAX - n) * K_EDGE,),
                                              a.dtype)
                seg = jnp.concatenate([seg, tail])
            segs.append(seg)
        return jnp.stack(segs).reshape(NW, NCH_MAX, K_EDGE)

    row_p = shard_edges(row, True)
    col_p = shard_edges(col, True)
    w_p = shard_edges(edge_weight, False)

    degp = _sc_deg(row_p, w_p)
    h0, u1a, u1b, dis = _tc_a(xs, degp.T, wp1, wq1, wr1, b1)

    v1a = _sc_prop(row_p, col_p, w_p, u1a)
    v1b = _sc_prop(row_p, col_p, w_p, u1b)
    t1, u2a, u2b = _tc_mid(v1a, v1b, dis)
    v2a = _sc_prop(row_p, col_p, w_p, u2a)
    v2b = _sc_prop(row_p, col_p, w_p, u2b)

    return _tc_b(h0, t1, v2a, v2b, dis, cw, cb, wp2, wq2, wr2, b2, bn,
                 l1w, l1b, l2w, l2b)


# speedup vs baseline: 1.3150x; 1.0182x over previous
import functools

import jax
import jax.numpy as jnp
import numpy as np
from jax.experimental import pallas as pl
from jax.experimental.pallas import tpu as pltpu
from jax.experimental.pallas import tpu_sc as plsc

N_NODES = 10000
C_IN = 128
HID = 64
W3 = 3 * HID
HW = W3 // 2
FEAT = 16
E_EDGES = 320000
SEQ = 5
BN_EPS = 1e-5
BLK = 1000

NC, NS, LANES = 2, 16, 16
NW = NC * NS
K_EDGE = 320
NCH0 = 32
NCH1 = 31
NCH_MAX = max(NCH0, NCH1)
E_CAP = NS * (NCH0 + NCH1) * K_EDGE
NPAD = 10112
KACC = 128
NCHA = NPAD // KACC
NPAD_DEG = 10240

_MESH = plsc.VectorSubcoreMesh(core_axis_name="c", subcore_axis_name="s")
_SC_PARAMS = pltpu.CompilerParams(use_tc_tiling_on_sc=False)


def _sigmoid(v):
    return 1.0 / (1.0 + jnp.exp(-v))


def _elu(v):
    return jnp.where(v > 0, v, jnp.exp(jnp.minimum(v, 0.0)) - 1.0)


@functools.partial(
    pl.kernel,
    out_type=jax.ShapeDtypeStruct((NC, NPAD_DEG), jnp.float32),
    mesh=_MESH,
    scratch_types=[
        pltpu.VMEM((NCH_MAX, K_EDGE), jnp.int32),
        pltpu.VMEM((NCH_MAX, K_EDGE), jnp.float32),
        pltpu.VMEM((640,), jnp.float32),
        pltpu.VMEM_SHARED((NPAD_DEG,), jnp.float32),
    ],
    compiler_params=_SC_PARAMS,
)
def _sc_deg(row_hbm, w_hbm, out_hbm, row_v, w_v, zbuf, acc):
    cid = jax.lax.axis_index("c")
    sid = jax.lax.axis_index("s")
    wid = sid * NC + cid
    pltpu.sync_copy(row_hbm.at[wid], row_v)
    pltpu.sync_copy(w_hbm.at[wid], w_v)

    def zb(i, carry):
        zbuf[pl.ds(i * LANES, LANES)] = jnp.zeros((LANES,), jnp.float32)
        return carry

    jax.lax.fori_loop(0, 640 // LANES, zb, None)
    pltpu.sync_copy(zbuf, acc.at[pl.ds(sid * 640, 640)])
    plsc.subcore_barrier()

    def body(j, carry):
        pltpu.sync_copy(w_v.at[j], acc.at[row_v.at[j]], add=True)
        return carry

    jax.lax.fori_loop(0, NCH_MAX, body, None)
    plsc.subcore_barrier()
    pltpu.sync_copy(acc.at[pl.ds(sid * 640, 640)],
                    out_hbm.at[cid, pl.ds(sid * 640, 640)])


@functools.partial(
    pl.kernel,
    out_type=jax.ShapeDtypeStruct((NC, NPAD, HW), jnp.float32),
    mesh=_MESH,
    scratch_types=[
        pltpu.VMEM((NCH_MAX, K_EDGE), jnp.int32),
        pltpu.VMEM((NCH_MAX, K_EDGE), jnp.int32),
        pltpu.VMEM((NCH_MAX, K_EDGE), jnp.float32),
        pltpu.VMEM((K_EDGE, HW), jnp.float32),
        pltpu.VMEM_SHARED((NPAD, HW), jnp.float32),
        pltpu.SemaphoreType.DMA,
    ],
    compiler_params=_SC_PARAMS,
)
def _sc_prop(row_hbm, col_hbm, w_hbm, src_hbm, part_hbm,
             row_v, col_v, w_v, rows, acc, sem):
    cid = jax.lax.axis_index("c")
    sid = jax.lax.axis_index("s")
    wid = sid * NC + cid
    pltpu.sync_copy(row_hbm.at[wid], row_v)
    pltpu.sync_copy(col_hbm.at[wid], col_v)
    pltpu.sync_copy(w_hbm.at[wid], w_v)

    def zrow(l, carry):
        for c in range(HW // LANES):
            rows[l, pl.ds(c * LANES, LANES)] = jnp.zeros((LANES,), jnp.float32)
        return carry

    jax.lax.fori_loop(0, KACC, zrow, None)
    for jj in range(5):
        j = sid + jj * NS

        @pl.when(j < NCHA)
        def _():
            pltpu.sync_copy(rows.at[pl.ds(0, KACC)],
                            acc.at[pl.ds(j * KACC, KACC)])

    plsc.subcore_barrier()

    def pb(j, carry):
        pltpu.async_copy(src_hbm.at[row_v.at[j]], rows, sem).wait()

        def sb(lg, c2):
            w16 = w_v[j, pl.ds(lg * LANES, LANES)]
            for e in range(LANES):
                we = jnp.full((LANES,), w16[e], jnp.float32)
                l = lg * LANES + e
                for c in range(HW // LANES):
                    sl = pl.ds(c * LANES, LANES)
                    rows[l, sl] = rows[l, sl] * we
            return c2

        jax.lax.fori_loop(0, K_EDGE // LANES, sb, None)
        pltpu.sync_copy(rows, acc.at[col_v.at[j]], add=True)
        return carry

    nch = jnp.where(cid == 0, NCH0, NCH1)
    jax.lax.fori_loop(0, nch, pb, None)
    plsc.subcore_barrier()
    for jj in range(5):
        j = sid + jj * NS

        @pl.when(j < NCHA)
        def _():
            pltpu.sync_copy(acc.at[pl.ds(j * KACC, KACC)],
                            part_hbm.at[cid, pl.ds(j * KACC, KACC)])


def _tc_a_body(x_ref, degp_ref, wp_ref, wq_ref, wr_ref, b_ref,
               h0_ref, ua_ref, ub_ref, dis_ref):
    xs = [x_ref[s] for s in range(SEQ)]
    for t in range(3):
        xt = jnp.concatenate([xs[t], xs[t + 1], xs[t + 2]], axis=1)
        p = jnp.dot(xt, wp_ref[...], preferred_element_type=jnp.float32) + b_ref[0]
        q = jnp.dot(xt, wq_ref[...], preferred_element_type=jnp.float32) + b_ref[1]
        r = jnp.dot(xt, wr_ref[...], preferred_element_type=jnp.float32) + b_ref[2]
        h0_ref[:, t * HID:(t + 1) * HID] = jnp.maximum(p * _sigmoid(q) + r, 0.0)
    deg = degp_ref[:, 0] + degp_ref[:, 1]
    good = deg > 0
    dis = jnp.where(good, jax.lax.rsqrt(jnp.where(good, deg, 1.0)), 0.0)
    dis_ref[...] = dis[:, None]
    h0 = h0_ref[...]
    ua_ref[...] = dis[:, None] * h0[:, :HW]
    ub_ref[...] = dis[:, None] * h0[:, HW:]


def _tc_a(xs, degp, wp, wq, wr, b):
    grid = N_NODES // BLK
    return pl.pallas_call(
        _tc_a_body,
        grid=(grid,),
        in_specs=[
            pl.BlockSpec((SEQ, BLK, C_IN), lambda i: (0, i, 0)),
            pl.BlockSpec((BLK, 2), lambda i: (i, 0)),
            pl.BlockSpec((3 * C_IN, HID), lambda i: (0, 0)),
            pl.BlockSpec((3 * C_IN, HID), lambda i: (0, 0)),
            pl.BlockSpec((3 * C_IN, HID), lambda i: (0, 0)),
            pl.BlockSpec((3, HID), lambda i: (0, 0)),
        ],
        out_specs=[
            pl.BlockSpec((BLK, W3), lambda i: (i, 0)),
            pl.BlockSpec((BLK, HW), lambda i: (i, 0)),
            pl.BlockSpec((BLK, HW), lambda i: (i, 0)),
            pl.BlockSpec((BLK, 1), lambda i: (i, 0)),
        ],
        out_shape=[
            jax.ShapeDtypeStruct((N_NODES, W3), jnp.float32),
            jax.ShapeDtypeStruct((N_NODES, HW), jnp.float32),
            jax.ShapeDtypeStruct((N_NODES, HW), jnp.float32),
            jax.ShapeDtypeStruct((N_NODES, 1), jnp.float32),
        ],
    )(xs, degp, wp, wq, wr, b)


def _tc_mid_body(va_ref, vb_ref, dis_ref, t1_ref, ua_ref, ub_ref):
    dis = dis_ref[...]
    ya = va_ref[0] + va_ref[1]
    yb = vb_ref[0] + vb_ref[1]
    t1a = -dis * ya
    t1b = -dis * yb
    t1_ref[:, :HW] = t1a
    t1_ref[:, HW:] = t1b
    ua_ref[...] = dis * t1a
    ub_ref[...] = dis * t1b


def _tc_mid(va, vb, dis):
    grid = N_NODES // BLK
    return pl.pallas_call(
        _tc_mid_body,
        grid=(grid,),
        in_specs=[
            pl.BlockSpec((NC, BLK, HW), lambda i: (0, i, 0)),
            pl.BlockSpec((NC, BLK, HW), lambda i: (0, i, 0)),
            pl.BlockSpec((BLK, 1), lambda i: (i, 0)),
        ],
        out_specs=[
            pl.BlockSpec((BLK, W3), lambda i: (i, 0)),
            pl.BlockSpec((BLK, HW), lambda i: (i, 0)),
            pl.BlockSpec((BLK, HW), lambda i: (i, 0)),
        ],
        out_shape=[
            jax.ShapeDtypeStruct((N_NODES, W3), jnp.float32),
            jax.ShapeDtypeStruct((N_NODES, HW), jnp.float32),
            jax.ShapeDtypeStruct((N_NODES, HW), jnp.float32),
        ],
    )(va, vb, dis)


def _tc_b_body(h0_ref, t1_ref, va_ref, vb_ref, dis_ref, cw_ref, cb_ref,
               wp_ref, wq_ref, wr_ref, b2_ref, bn_ref,
               l1w_ref, l1b_ref, l2w_ref, l2b_ref, out_ref):
    h0 = h0_ref[...]
    t1 = t1_ref[...]
    dis = dis_ref[...]
    p2a = -dis * (va_ref[0] + va_ref[1])
    p2b = -dis * (vb_ref[0] + vb_ref[1])
    p2 = jnp.concatenate([p2a, p2b], axis=1)
    gs = []
    for t in range(3):
        sl = slice(t * HID, (t + 1) * HID)
        tx0 = h0[:, sl]
        tx2 = 2.0 * p2[:, sl] - tx0
        o = (jnp.dot(tx0, cw_ref[0], preferred_element_type=jnp.float32)
             + jnp.dot(t1[:, sl], cw_ref[1], preferred_element_type=jnp.float32)
             + jnp.dot(tx2, cw_ref[2], preferred_element_type=jnp.float32)
             + cb_ref[...])
        gs.append(jnp.maximum(o, 0.0))
    g = jnp.concatenate(gs, axis=1)
    p = jnp.dot(g, wp_ref[...], preferred_element_type=jnp.float32) + b2_ref[0]
    q = jnp.dot(g, wq_ref[...], preferred_element_type=jnp.float32) + b2_ref[1]
    r = jnp.dot(g, wr_ref[...], preferred_element_type=jnp.float32) + b2_ref[2]
    f = jnp.maximum(p * _sigmoid(q) + r, 0.0)
    f = f * bn_ref[:, 0:1] + bn_ref[:, 1:2]
    k = _elu(f)
    k = jnp.dot(k, l1w_ref[...], preferred_element_type=jnp.float32) + l1b_ref[...]
    k = _elu(k)
    out_ref[...] = jnp.dot(k, l2w_ref[...], preferred_element_type=jnp.float32) \
        + l2b_ref[...]


def _tc_b(h0, t1, va, vb, dis, cw, cb, wp, wq, wr, b2, bn, l1w, l1b, l2w, l2b):
    grid = N_NODES // BLK
    return pl.pallas_call(
        _tc_b_body,
        grid=(grid,),
        in_specs=[
            pl.BlockSpec((BLK, W3), lambda i: (i, 0)),
            pl.BlockSpec((BLK, W3), lambda i: (i, 0)),
            pl.BlockSpec((NC, BLK, HW), lambda i: (0, i, 0)),
            pl.BlockSpec((NC, BLK, HW), lambda i: (0, i, 0)),
            pl.BlockSpec((BLK, 1), lambda i: (i, 0)),
            pl.BlockSpec((3, HID, HID), lambda i: (0, 0, 0)),
            pl.BlockSpec((1, HID), lambda i: (0, 0)),
            pl.BlockSpec((W3, FEAT), lambda i: (0, 0)),
            pl.BlockSpec((W3, FEAT), lambda i: (0, 0)),
            pl.BlockSpec((W3, FEAT), lambda i: (0, 0)),
            pl.BlockSpec((3, FEAT), lambda i: (0, 0)),
            pl.BlockSpec((BLK, 2), lambda i: (i, 0)),
            pl.BlockSpec((FEAT, 10), lambda i: (0, 0)),
            pl.BlockSpec((1, 10), lambda i: (0, 0)),
            pl.BlockSpec((10, 1), lambda i: (0, 0)),
            pl.BlockSpec((1, 1), lambda i: (0, 0)),
        ],
        out_specs=pl.BlockSpec((BLK, 1), lambda i: (i, 0)),
        out_shape=jax.ShapeDtypeStruct((N_NODES, 1), jnp.float32),
    )(h0, t1, va, vb, dis, cw, cb, wp, wq, wr, b2, bn, l1w, l1b, l2w, l2b)


def kernel(x, edge_index, edge_weight, params):
    row, col = edge_index[0], edge_index[1]
    xs = x[0]

    def tconv_w(w):
        return jnp.transpose(w[:, :, 0, :], (2, 1, 0)).reshape(w.shape[1] * 3,
                                                               w.shape[0])

    wp1, wq1, wr1 = (tconv_w(w) for w in params["tc1_w"])
    b1 = jnp.stack(params["tc1_b"])
    wp2, wq2, wr2 = (tconv_w(w) for w in params["tc2_w"])
    b2 = jnp.stack(params["tc2_b"])
    cw = jnp.stack(params["cheb_w"])
    cb = params["cheb_b"][None, :]
    bn_scale = params["bn_w"] * np.float32(1.0 / np.sqrt(1.0 + BN_EPS))
    bn = jnp.stack([bn_scale, params["bn_b"]], axis=1)
    l1w = params["lin1_w"].T
    l1b = params["lin1_b"][None, :]
    l2w = params["lin2_w"].T
    l2b = params["lin2_b"][None, :]

    spread = (jnp.arange(NCH_MAX * K_EDGE, dtype=jnp.int32) * 8) % N_NODES

    def shard_edges(a, idx_pad):
        pad = spread[:E_CAP - E_EDGES].astype(a.dtype) if idx_pad \
            else jnp.zeros((E_CAP - E_EDGES,), a.dtype)
        a = jnp.concatenate([a, pad])
        segs, off = [], 0
        for w in range(NW):
            n = NCH0 if (w % NC == 0) else NCH1
            s = n * K_EDGE
            seg = a[off:off + s]
            off += s
            if n < NCH_MAX:
                tail = spread[:(NCH_MAX - n) * K_EDGE].astype(a.dtype) \
                    if idx_pad else jnp.zeros(((NCH_MAX - n) * K_EDGE,),
                                              a.dtype)
                seg = jnp.concatenate([seg, tail])
            segs.append(seg)
        return jnp.stack(segs).reshape(NW, NCH_MAX, K_EDGE)

    row_p = shard_edges(row, True)
    col_p = shard_edges(col, True)
    w_p = shard_edges(edge_weight, False)

    degp = _sc_deg(row_p, w_p)
    h0, u1a, u1b, dis = _tc_a(xs, degp.T, wp1, wq1, wr1, b1)

    v1a = _sc_prop(row_p, col_p, w_p, u1a)
    v1b = _sc_prop(row_p, col_p, w_p, u1b)
    t1, u2a, u2b = _tc_mid(v1a, v1b, dis)
    v2a = _sc_prop(row_p, col_p, w_p, u2a)
    v2b = _sc_prop(row_p, col_p, w_p, u2b)

    return _tc_b(h0, t1, v2a, v2b, dis, cw, cb, wp2, wq2, wr2, b2, bn,
                 l1w, l1b, l2w, l2b)
